# baseline (speedup 1.0000x reference)
import functools

import jax
import jax.numpy as jnp
from jax import lax
from jax.experimental import pallas as pl
from jax.experimental.pallas import tpu as pltpu
from jax.experimental.pallas import tpu_sc as plsc

_B = 64
_N = 100000
_RG = 8


_GUMBEL_CACHE = []


def _gumbel_const():
    if not _GUMBEL_CACHE:
        g = jax.random.gumbel(jax.random.key(42), (_B, _N), jnp.float32)
        _GUMBEL_CACHE.append(jax.block_until_ready(g))
    return _GUMBEL_CACHE[0]


def _tc_body(logits_ref, gum_ref, flat_idx_ref, logp_ref):
    logits = logits_ref[...]
    x = logits + gum_ref[...]
    m = jnp.max(logits, axis=1, keepdims=True)
    s = jnp.sum(jnp.exp(logits - m), axis=1, keepdims=True)
    lse = m + jnp.log(s)
    xm = jnp.max(x, axis=1, keepdims=True)
    ii = lax.broadcasted_iota(jnp.int32, (_RG, _N), 1)
    first = jnp.min(jnp.where(x == xm, ii, _N), axis=1, keepdims=True)
    chosen = jnp.sum(jnp.where(ii == first, logits, 0.0), axis=1, keepdims=True)
    rows = pl.program_id(0) * _RG + lax.broadcasted_iota(jnp.int32, (_RG, 1), 0)
    flat_idx_ref[...] = jnp.broadcast_to(rows * _N + first, (_RG, 128))
    logp_ref[...] = jnp.broadcast_to(chosen - lse, (_RG, 128))


def _tc_stats(logits, gumbel):
    return pl.pallas_call(
        _tc_body,
        grid=(_B // _RG,),
        in_specs=[
            pl.BlockSpec((_RG, _N), lambda i: (i, 0)),
            pl.BlockSpec((_RG, _N), lambda i: (i, 0)),
        ],
        out_specs=[
            pl.BlockSpec((_RG, 128), lambda i: (i, 0)),
            pl.BlockSpec((_RG, 128), lambda i: (i, 0)),
        ],
        out_shape=[
            jax.ShapeDtypeStruct((_B, 128), jnp.int32),
            jax.ShapeDtypeStruct((_B, 128), jnp.float32),
        ],
    )(logits, gumbel)


def _sc_gather_body(flat_hbm, idx_hbm, out_hbm, idx_v, vals_v, sem):
    wid = lax.axis_index("s") * 2 + lax.axis_index("c")

    @pl.when(wid == 0)
    def _():
        pltpu.sync_copy(idx_hbm, idx_v)
        pltpu.async_copy(flat_hbm.at[idx_v], vals_v, sem).wait()
        pltpu.sync_copy(vals_v, out_hbm)


@functools.cache
def _sc_gather():
    return pl.kernel(
        _sc_gather_body,
        out_type=jax.ShapeDtypeStruct((_B,), jnp.int32),
        mesh=plsc.VectorSubcoreMesh(core_axis_name="c", subcore_axis_name="s"),
        scratch_types=[
            pltpu.VMEM((_B,), jnp.int32),
            pltpu.VMEM((_B,), jnp.int32),
            pltpu.SemaphoreType.DMA,
        ],
    )


def kernel(all_logits_list, all_action_inds_list):
    gumbel = _gumbel_const()
    flat_idx, logp = _tc_stats(all_logits_list, gumbel)
    actions = _sc_gather()(all_action_inds_list.reshape(-1), flat_idx[:, 0])
    return actions, logp[:, 0]
